# Initial kernel scaffold; baseline (speedup 1.0000x reference)
import jax
import jax.numpy as jnp
from jax.experimental import pallas as pl


def kernel(x, edge_index, params):
    raise NotImplementedError("write your pallas kernel here")


# speedup vs baseline: 31.6123x; 31.6123x over previous
import functools

import jax
import jax.numpy as jnp
from jax import lax
from jax.experimental import pallas as pl
from jax.experimental.pallas import tpu as pltpu
from jax.experimental.pallas import tpu_sc as plsc

N = 100000
D = 16
E = 6400000
NC, NS = 2, 16
NW = NC * NS
LANE = 128
RPC = 8
CHUNK = RPC * LANE
NROW = E // LANE
NCHUNK = NROW // RPC
MAXIT = (NCHUNK + NW - 1) // NW
RPT = N // NS
ZR = 125

_mesh = plsc.VectorSubcoreMesh(core_axis_name="c", subcore_axis_name="s",
                               num_cores=NC)


@functools.partial(
    pl.kernel,
    out_type=jax.ShapeDtypeStruct((NC, NS, RPT, D), jnp.float32),
    mesh=_mesh,
    scratch_types=[
        pltpu.VMEM((RPC, LANE), jnp.int32),
        pltpu.VMEM((RPC, LANE), jnp.int32),
        pltpu.VMEM((CHUNK, D), jnp.float32),
        pltpu.VMEM((ZR, D), jnp.float32),
        pltpu.VMEM_SHARED((N, D), jnp.float32),
        pltpu.SemaphoreType.DMA,
    ],
    compiler_params=pltpu.CompilerParams(use_tc_tiling_on_sc=False),
)
def _sc_agg(h_hbm, src_hbm, dst_hbm, out_hbm, sidx, didx, msgs, zbuf, acc,
            gsem):
    cid = lax.axis_index("c")
    sid = lax.axis_index("s")
    wid = sid * NC + cid

    def _zrow(i, carry):
        zbuf[i, :] = jnp.zeros((D,), jnp.float32)
        return carry
    lax.fori_loop(0, ZR, _zrow, 0)
    for k in range(RPT // ZR):
        pltpu.sync_copy(zbuf, acc.at[pl.ds(sid * RPT + k * ZR, ZR)])
    plsc.subcore_barrier()

    def _chunk(i, carry):
        c = wid + i * NW

        @pl.when(c < NCHUNK)
        def _():
            row0 = c * RPC
            pltpu.sync_copy(src_hbm.at[pl.ds(row0, RPC)], sidx)
            pltpu.sync_copy(dst_hbm.at[pl.ds(row0, RPC)], didx)
            cps = []
            for j in range(RPC):
                cps.append(pltpu.async_copy(
                    h_hbm.at[sidx.at[j]],
                    msgs.at[pl.ds(j * LANE, LANE)], gsem))
            for cp in cps:
                cp.wait()
            for j in range(RPC):
                pltpu.sync_copy(msgs.at[pl.ds(j * LANE, LANE)],
                                acc.at[didx.at[j]], add=True)
        return carry

    lax.fori_loop(0, MAXIT, _chunk, 0)
    plsc.subcore_barrier()

    pltpu.sync_copy(acc.at[pl.ds(sid * RPT, RPT)], out_hbm.at[cid, sid])


RB = 2000
GB = N // RB


def _dense_body(scale_ref, hin_ref, a0_ref, a1_ref, wa_ref, ba_ref, wb_ref,
                bb_ref, wr_ref, br_ref, u_ref, st_ref, *, alpha):
    i = pl.program_id(0)
    hin = hin_ref[...]
    t = scale_ref[0, 0] * hin + a0_ref[...] + a1_ref[...]
    z = jnp.dot(t, wa_ref[...], preferred_element_type=jnp.float32) \
        + ba_ref[...]
    z = jnp.where(z > 0, z, alpha * z)
    u = jnp.dot(z, wb_ref[...], preferred_element_type=jnp.float32) \
        + bb_ref[...]
    u = u + jnp.dot(hin, wr_ref[...], preferred_element_type=jnp.float32) \
        + br_ref[...]
    u_ref[...] = u

    @pl.when(i == 0)
    def _():
        st_ref[...] = jnp.zeros_like(st_ref)

    st_ref[...] += jnp.stack([jnp.sum(u, axis=0), jnp.sum(u * u, axis=0)])


def _dense(hin, a0, a1, scale, wa, ba, wb, bb, wr, br, alpha):
    row = lambda i: (i, 0)
    fixed = lambda i: (0, 0)
    return pl.pallas_call(
        functools.partial(_dense_body, alpha=alpha),
        grid=(GB,),
        in_specs=[
            pl.BlockSpec(memory_space=pltpu.SMEM),
            pl.BlockSpec((RB, D), row),
            pl.BlockSpec((RB, D), row),
            pl.BlockSpec((RB, D), row),
            pl.BlockSpec((D, D), fixed),
            pl.BlockSpec((1, D), fixed),
            pl.BlockSpec((D, D), fixed),
            pl.BlockSpec((1, D), fixed),
            pl.BlockSpec((D, D), fixed),
            pl.BlockSpec((1, D), fixed),
        ],
        out_specs=[
            pl.BlockSpec((RB, D), row),
            pl.BlockSpec((2, D), fixed),
        ],
        out_shape=[
            jax.ShapeDtypeStruct((N, D), jnp.float32),
            jax.ShapeDtypeStruct((2, D), jnp.float32),
        ],
    )(scale, hin, a0, a1, wa, ba, wb, bb, wr, br)


def _bn_body(u_ref, st_ref, g_ref, b_ref, o_ref):
    m = st_ref[0, :] * (1.0 / N)
    ex2 = st_ref[1, :] * (1.0 / N)
    inv = lax.rsqrt(ex2 - m * m + 1e-5)
    a = inv * g_ref[0, :]
    c = b_ref[0, :] - m * a
    o_ref[...] = u_ref[...] * a[None, :] + c[None, :]


def _bn_apply(u, st, g, b):
    row = lambda i: (i, 0)
    fixed = lambda i: (0, 0)
    return pl.pallas_call(
        _bn_body,
        grid=(GB,),
        in_specs=[
            pl.BlockSpec((RB, D), row),
            pl.BlockSpec((2, D), fixed),
            pl.BlockSpec((1, D), fixed),
            pl.BlockSpec((1, D), fixed),
        ],
        out_specs=pl.BlockSpec((RB, D), row),
        out_shape=jax.ShapeDtypeStruct((N, D), jnp.float32),
    )(u, st, g.reshape(1, D), b.reshape(1, D))


def _layer(h, src, dst, scale, wa, ba, wb, bb, wr, br, g, be, alpha):
    agg = _sc_agg(h, src, dst).reshape(NC, N, D)
    u, st = _dense(h, agg[0], agg[1], scale.reshape(1, 1),
                   wa, ba.reshape(1, D), wb, bb.reshape(1, D),
                   wr, br.reshape(1, D), alpha)
    return _bn_apply(u, st, g, be)


def kernel(x, edge_index, params):
    p = params
    ei = edge_index.astype(jnp.int32)
    src = ei[0].reshape(NROW, LANE)
    dst = ei[1].reshape(NROW, LANE)
    din = x.shape[1]
    x16 = jnp.pad(x, ((0, 0), (0, D - din)))
    w1a = jnp.pad(p['W1a'], ((0, D - din), (0, 0)))
    wr1 = jnp.pad(p['Wr1'], ((0, D - din), (0, 0)))

    h = _layer(x16, src, dst, 1.0 + p['eps1'], w1a, p['b1a'], p['W1b'],
               p['b1b'], wr1, p['br1'], p['g1'], p['be1'], alpha=0.01)
    h = _layer(h, src, dst, 1.0 + p['eps2'], p['W2a'], p['b2a'], p['W2b'],
               p['b2b'], p['Wr2'], p['br2'], p['g2'], p['be2'], alpha=0.0)
    h = _layer(h, src, dst, 1.0 + p['eps3'], p['W3a'], p['b3a'], p['W3b'],
               p['b3b'], p['Wr3'], p['br3'], p['g3'], p['be3'], alpha=0.0)
    return h


# speedup vs baseline: 35.2419x; 1.1148x over previous
import functools

import jax
import jax.numpy as jnp
import numpy as np
from jax import lax
from jax.experimental import pallas as pl
from jax.experimental.pallas import tpu as pltpu
from jax.experimental.pallas import tpu_sc as plsc

N = 100000
D = 16
E = 6400000
NC, NS = 2, 16
NW = NC * NS
LANE = 128
RPC = 8
CHUNK = RPC * LANE
NROW = E // LANE
NCHUNK = NROW // RPC
MAXIT = (NCHUNK + NW - 1) // NW
RPT = N // NS
ZR = 125
NP = N // 8

_mesh = plsc.VectorSubcoreMesh(core_axis_name="c", subcore_axis_name="s",
                               num_cores=NC)


@functools.partial(
    pl.kernel,
    out_type=jax.ShapeDtypeStruct((NC, NS, RPT, D), jnp.float32),
    mesh=_mesh,
    scratch_types=[
        pltpu.VMEM((RPC, LANE), jnp.int32),
        pltpu.VMEM((RPC, LANE), jnp.int32),
        pltpu.VMEM((CHUNK, D), jnp.float32),
        pltpu.VMEM((ZR, D), jnp.float32),
        pltpu.VMEM_SHARED((N, D), jnp.float32),
        pltpu.SemaphoreType.DMA,
    ],
    compiler_params=pltpu.CompilerParams(use_tc_tiling_on_sc=False),
)
def _sc_agg(h_hbm, src_hbm, dst_hbm, out_hbm, sidx, didx, msgs, zbuf, acc,
            gsem):
    cid = lax.axis_index("c")
    sid = lax.axis_index("s")
    wid = sid * NC + cid

    def _zrow(i, carry):
        zbuf[i, :] = jnp.zeros((D,), jnp.float32)
        return carry
    lax.fori_loop(0, ZR, _zrow, 0)
    for k in range(RPT // ZR):
        pltpu.sync_copy(zbuf, acc.at[pl.ds(sid * RPT + k * ZR, ZR)])
    plsc.subcore_barrier()

    def _chunk(i, carry):
        c = wid + i * NW

        @pl.when(c < NCHUNK)
        def _():
            row0 = c * RPC
            pltpu.sync_copy(src_hbm.at[pl.ds(row0, RPC)], sidx)
            pltpu.sync_copy(dst_hbm.at[pl.ds(row0, RPC)], didx)
            cps = []
            for j in range(RPC):
                cps.append(pltpu.async_copy(
                    h_hbm.at[sidx.at[j]],
                    msgs.at[pl.ds(j * LANE, LANE)], gsem))
            for cp in cps:
                cp.wait()
            for j in range(RPC):
                pltpu.sync_copy(msgs.at[pl.ds(j * LANE, LANE)],
                                acc.at[didx.at[j]], add=True)
        return carry

    lax.fori_loop(0, MAXIT, _chunk, 0)
    plsc.subcore_barrier()

    pltpu.sync_copy(acc.at[pl.ds(sid * RPT, RPT)], out_hbm.at[cid, sid])


def _make_layer_kernel(alpha):
    def body(scale_ref, hin_ref, a0_ref, a1_ref, wa_ref, ba_ref, wb_ref,
             bb_ref, wr_ref, br_ref, g_ref, be_ref, p_ref, o_ref, u_scr,
             st_scr):
        ph = pl.program_id(0)

        @pl.when(ph == 0)
        def _():
            hin = hin_ref[...]
            t = scale_ref[0, 0] * hin + a0_ref[...] + a1_ref[...]
            z = jnp.dot(t, wa_ref[...], preferred_element_type=jnp.float32, precision=lax.Precision.HIGHEST) \
                + ba_ref[...]
            z = jnp.where(z > 0, z, alpha * z)
            u = jnp.dot(z, wb_ref[...], preferred_element_type=jnp.float32, precision=lax.Precision.HIGHEST) \
                + bb_ref[...]
            u = u + jnp.dot(hin, wr_ref[...],
                            preferred_element_type=jnp.float32, precision=lax.Precision.HIGHEST) + br_ref[...]
            u_scr[...] = u
            st_scr[...] = jnp.stack(
                [jnp.sum(u, axis=0), jnp.sum(u * u, axis=0)])

        @pl.when(ph == 1)
        def _():
            sf = jnp.dot(st_scr[...], p_ref[...],
                         preferred_element_type=jnp.float32, precision=lax.Precision.HIGHEST)
            m = sf[0:1, :] * (1.0 / N)
            ex2 = sf[1:2, :] * (1.0 / N)
            inv = lax.rsqrt(ex2 - m * m + 1e-5)
            a = inv * g_ref[...]
            c = be_ref[...] - m * a
            o_ref[...] = u_scr[...] * a + c

    full = lambda i: (0, 0)
    return pl.pallas_call(
        body,
        grid=(2,),
        in_specs=[
            pl.BlockSpec(memory_space=pltpu.SMEM),
            pl.BlockSpec((NP, 128), full),
            pl.BlockSpec((NP, 128), full),
            pl.BlockSpec((NP, 128), full),
            pl.BlockSpec((128, 128), full),
            pl.BlockSpec((1, 128), full),
            pl.BlockSpec((128, 128), full),
            pl.BlockSpec((1, 128), full),
            pl.BlockSpec((128, 128), full),
            pl.BlockSpec((1, 128), full),
            pl.BlockSpec((1, 128), full),
            pl.BlockSpec((1, 128), full),
            pl.BlockSpec((128, 128), full),
        ],
        out_specs=pl.BlockSpec((NP, 128), full),
        out_shape=jax.ShapeDtypeStruct((NP, 128), jnp.float32),
        scratch_shapes=[
            pltpu.VMEM((NP, 128), jnp.float32),
            pltpu.VMEM((2, 128), jnp.float32),
        ],
    )


_layer_leaky = _make_layer_kernel(0.01)
_layer_relu = _make_layer_kernel(0.0)


def _blockdiag(w):
    z = jnp.zeros((8, D, 8, D), jnp.float32)
    i = jnp.arange(8)
    z = z.at[i, :, i, :].set(jnp.broadcast_to(w, (8, D, D)))
    return z.reshape(128, 128)


def _tile8(b):
    return jnp.tile(b.reshape(1, D), (1, 8)).reshape(1, 128)


_lane = np.arange(128)
_FOLD = np.asarray(
    (_lane[:, None] % D) == (_lane[None, :] % D), dtype=np.float32)


def _layer(h128, src, dst, scale, wa, ba, wb, bb, wr, br, g, be, leaky):
    agg = _sc_agg(h128.reshape(N, D), src, dst)
    agg = agg.reshape(NC, NP, 128)
    f = _layer_leaky if leaky else _layer_relu
    return f(scale.reshape(1, 1), h128, agg[0], agg[1],
             _blockdiag(wa), _tile8(ba), _blockdiag(wb), _tile8(bb),
             _blockdiag(wr), _tile8(br), _tile8(g), _tile8(be), _FOLD)


def kernel(x, edge_index, params):
    p = params
    ei = edge_index.astype(jnp.int32)
    src = ei[0].reshape(NROW, LANE)
    dst = ei[1].reshape(NROW, LANE)
    din = x.shape[1]
    x128 = jnp.pad(x, ((0, 0), (0, D - din))).reshape(NP, 128)
    w1a = jnp.pad(p['W1a'], ((0, D - din), (0, 0)))
    wr1 = jnp.pad(p['Wr1'], ((0, D - din), (0, 0)))

    h = _layer(x128, src, dst, 1.0 + p['eps1'], w1a, p['b1a'], p['W1b'],
               p['b1b'], wr1, p['br1'], p['g1'], p['be1'], leaky=True)
    h = _layer(h, src, dst, 1.0 + p['eps2'], p['W2a'], p['b2a'], p['W2b'],
               p['b2b'], p['Wr2'], p['br2'], p['g2'], p['be2'], leaky=False)
    h = _layer(h, src, dst, 1.0 + p['eps3'], p['W3a'], p['b3a'], p['W3b'],
               p['b3b'], p['Wr3'], p['br3'], p['g3'], p['be3'], leaky=False)
    return h.reshape(N, D)


# speedup vs baseline: 37.9365x; 1.0765x over previous
import functools

import jax
import jax.numpy as jnp
import numpy as np
from jax import lax
from jax.experimental import pallas as pl
from jax.experimental.pallas import tpu as pltpu
from jax.experimental.pallas import tpu_sc as plsc

N = 100000
D = 16
E = 6400000
NC, NS = 2, 16
NW = NC * NS
LANE = 128
RPC = 8
CHUNK = RPC * LANE
NROW = E // LANE
NCHUNK = NROW // RPC
MAXIT = (NCHUNK + NW - 1) // NW
RPT = N // NS
ZR = 125
NP = N // 8
CH = 1250

_mesh = plsc.VectorSubcoreMesh(core_axis_name="c", subcore_axis_name="s",
                               num_cores=NC)


@functools.partial(
    pl.kernel,
    out_type=jax.ShapeDtypeStruct((NC, NS, RPT, D), jnp.float32),
    mesh=_mesh,
    scratch_types=[
        pltpu.VMEM((RPC, LANE), jnp.int32),
        pltpu.VMEM((RPC, LANE), jnp.int32),
        pltpu.VMEM((CHUNK, D), jnp.float32),
        pltpu.VMEM((ZR, D), jnp.float32),
        pltpu.VMEM_SHARED((N, D), jnp.float32),
        pltpu.SemaphoreType.DMA,
    ],
    compiler_params=pltpu.CompilerParams(use_tc_tiling_on_sc=False),
)
def _sc_agg(h_hbm, src_hbm, dst_hbm, out_hbm, sidx, didx, msgs, zbuf, acc,
            gsem):
    cid = lax.axis_index("c")
    sid = lax.axis_index("s")
    wid = sid * NC + cid

    def _zrow(i, carry):
        zbuf[i, :] = jnp.zeros((D,), jnp.float32)
        return carry
    lax.fori_loop(0, ZR, _zrow, 0)
    for k in range(RPT // ZR):
        pltpu.sync_copy(zbuf, acc.at[pl.ds(sid * RPT + k * ZR, ZR)])
    plsc.subcore_barrier()

    def _chunk(i, carry):
        c = wid + i * NW

        @pl.when(c < NCHUNK)
        def _():
            row0 = c * RPC
            pltpu.sync_copy(src_hbm.at[pl.ds(row0, RPC)], sidx)
            pltpu.sync_copy(dst_hbm.at[pl.ds(row0, RPC)], didx)
            cps = []
            for j in range(RPC):
                cps.append(pltpu.async_copy(
                    h_hbm.at[sidx.at[j]],
                    msgs.at[pl.ds(j * LANE, LANE)], gsem))
            for cp in cps:
                cp.wait()
            for j in range(RPC):
                pltpu.sync_copy(msgs.at[pl.ds(j * LANE, LANE)],
                                acc.at[didx.at[j]], add=True)
        return carry

    lax.fori_loop(0, MAXIT, _chunk, 0)
    plsc.subcore_barrier()

    pltpu.sync_copy(acc.at[pl.ds(sid * RPT, RPT)], out_hbm.at[cid, sid])


def _make_layer_kernel(alpha):
    def body(scale_ref, hin_ref, a0_ref, a1_ref, wa_ref, ba_ref, wb_ref,
             bb_ref, wr_ref, br_ref, g_ref, be_ref, p_ref, o_ref, u_scr,
             st_scr):
        ph = pl.program_id(0)

        @pl.when(ph == 0)
        def _():
            s1 = jnp.zeros((1, 128), jnp.float32)
            s2 = jnp.zeros((1, 128), jnp.float32)
            for k in range(NP // CH):
                sl = pl.ds(k * CH, CH)
                hin = hin_ref[sl, :]
                t = scale_ref[0, 0] * hin + a0_ref[sl, :] + a1_ref[sl, :]
                bf = jnp.bfloat16
                z = jnp.dot(t.astype(bf), wa_ref[...],
                            preferred_element_type=jnp.float32) + ba_ref[...]
                z = jnp.where(z > 0, z, alpha * z)
                u = jnp.dot(z.astype(bf), wb_ref[...],
                            preferred_element_type=jnp.float32) + bb_ref[...]
                u = u + jnp.dot(hin.astype(bf), wr_ref[...],
                                preferred_element_type=jnp.float32) \
                    + br_ref[...]
                u_scr[sl, :] = u
                s1 = s1 + jnp.sum(u, axis=0, keepdims=True)
            st_scr[0:1, :] = s1
            st_scr[1:2, :] = s1

        @pl.when(ph == 1)
        def _():
            m = jnp.dot(st_scr[0:1, :], p_ref[...],
                        preferred_element_type=jnp.float32,
                        precision=lax.Precision.HIGHEST) * (1.0 / N)
            sv = jnp.zeros((1, 128), jnp.float32)
            for k in range(NP // CH):
                du = u_scr[pl.ds(k * CH, CH), :] - m
                sv = sv + jnp.sum(du * du, axis=0, keepdims=True)
            v = jnp.dot(sv, p_ref[...],
                        preferred_element_type=jnp.float32,
                        precision=lax.Precision.HIGHEST) * (1.0 / N)
            inv = 1.0 / jnp.sqrt(v + 1e-5)
            a = inv * g_ref[...]
            c = be_ref[...] - m * a
            o_ref[...] = u_scr[...] * a + c

    full = lambda i: (0, 0)
    return pl.pallas_call(
        body,
        grid=(2,),
        in_specs=[
            pl.BlockSpec(memory_space=pltpu.SMEM),
            pl.BlockSpec((NP, 128), full),
            pl.BlockSpec((NP, 128), full),
            pl.BlockSpec((NP, 128), full),
            pl.BlockSpec((128, 128), full),
            pl.BlockSpec((1, 128), full),
            pl.BlockSpec((128, 128), full),
            pl.BlockSpec((1, 128), full),
            pl.BlockSpec((128, 128), full),
            pl.BlockSpec((1, 128), full),
            pl.BlockSpec((1, 128), full),
            pl.BlockSpec((1, 128), full),
            pl.BlockSpec((128, 128), full),
        ],
        out_specs=pl.BlockSpec((NP, 128), full),
        out_shape=jax.ShapeDtypeStruct((NP, 128), jnp.float32),
        scratch_shapes=[
            pltpu.VMEM((NP, 128), jnp.float32),
            pltpu.VMEM((2, 128), jnp.float32),
        ],
    )


_layer_leaky = _make_layer_kernel(0.01)
_layer_relu = _make_layer_kernel(0.0)


def _blockdiag(w):
    z = jnp.zeros((8, D, 8, D), jnp.float32)
    i = jnp.arange(8)
    z = z.at[i, :, i, :].set(jnp.broadcast_to(w, (8, D, D)))
    return z.reshape(128, 128)


def _tile8(b):
    return jnp.tile(b.reshape(1, D), (1, 8)).reshape(1, 128)


_lane = np.arange(128)
_FOLD = np.asarray(
    (_lane[:, None] % D) == (_lane[None, :] % D), dtype=np.float32)


def _layer(h128, src, dst, scale, wa, ba, wb, bb, wr, br, g, be, leaky):
    agg = _sc_agg(h128.reshape(N, D), src, dst)
    agg = agg.reshape(NC, NP, 128)
    f = _layer_leaky if leaky else _layer_relu
    bf = jnp.bfloat16
    return f(scale.reshape(1, 1), h128, agg[0], agg[1],
             _blockdiag(wa).astype(bf), _tile8(ba),
             _blockdiag(wb).astype(bf), _tile8(bb),
             _blockdiag(wr).astype(bf), _tile8(br),
             _tile8(g), _tile8(be), _FOLD)


def kernel(x, edge_index, params):
    p = params
    ei = edge_index.astype(jnp.int32)
    src = ei[0].reshape(NROW, LANE)
    dst = ei[1].reshape(NROW, LANE)
    din = x.shape[1]
    x128 = jnp.pad(x, ((0, 0), (0, D - din))).reshape(NP, 128)
    w1a = jnp.pad(p['W1a'], ((0, D - din), (0, 0)))
    wr1 = jnp.pad(p['Wr1'], ((0, D - din), (0, 0)))

    h = _layer(x128, src, dst, 1.0 + p['eps1'], w1a, p['b1a'], p['W1b'],
               p['b1b'], wr1, p['br1'], p['g1'], p['be1'], leaky=True)
    h = _layer(h, src, dst, 1.0 + p['eps2'], p['W2a'], p['b2a'], p['W2b'],
               p['b2b'], p['Wr2'], p['br2'], p['g2'], p['be2'], leaky=False)
    h = _layer(h, src, dst, 1.0 + p['eps3'], p['W3a'], p['b3a'], p['W3b'],
               p['b3b'], p['Wr3'], p['br3'], p['g3'], p['be3'], leaky=False)
    return h.reshape(N, D)
